# baseline (speedup 1.0000x reference)
import functools

import jax
import jax.numpy as jnp
from jax import lax
from jax.experimental import pallas as pl
from jax.experimental.pallas import tpu as pltpu
from jax.experimental.pallas import tpu_sc as plsc

_VOCAB = 100000
_D = 64
_S = 200
_SP = 256
_B = 1024
_H = _S // 2

_NC = 2
_NS = 16
_NW = _NC * _NS
_SPW = _B // _NW
_NB = 8
_LA = 4

_SEQ_BLK = 32

_SPLITS = ((0, 104), (104, 96))


def _gather_body(table_hbm, idx_hbm, pos_hbm, out_hbm,
                 idx_v, pos_v, bufs, gsems, ssems):
    wid = lax.axis_index("s") * _NC + lax.axis_index("c")
    sbase = wid * _SPW

    pltpu.sync_copy(pos_hbm, pos_v)
    pltpu.sync_copy(idx_hbm.at[pl.ds(wid * _SPW, _SPW), :], idx_v)

    def start_gather(s, b):
        buf = bufs[b]
        for o, n in _SPLITS:
            pltpu.make_async_copy(
                table_hbm.at[idx_v.at[s, pl.ds(o, n)]],
                buf.at[pl.ds(o, n)], gsems[b]).start()

    def wait_gather(s, b):
        buf = bufs[b]
        for o, n in _SPLITS:
            pltpu.make_async_copy(
                table_hbm.at[idx_v.at[s, pl.ds(o, n)]],
                buf.at[pl.ds(o, n)], gsems[b]).wait()

    def start_store(s, b):
        buf = bufs[b]
        r0 = (sbase + s) * _H
        for h in range(2):
            pltpu.make_async_copy(
                buf.at[pl.ds(h * _H, _H), :],
                out_hbm.at[pl.ds(r0, _H), pl.ds(h * _D, _D)], ssems[b]).start()

    def wait_store(b):
        buf = bufs[b]
        for h in range(2):
            pltpu.make_async_copy(
                buf.at[pl.ds(h * _H, _H), :],
                out_hbm.at[pl.ds(0, _H), pl.ds(h * _D, _D)], ssems[b]).wait()

    def step(s, b, issue_next, wstore):
        if issue_next:
            nb = (b + _LA) % _NB
            if wstore:
                wait_store(nb)
            start_gather(s + _LA, nb)
        wait_gather(s, b)
        buf = bufs[b]

        def add_rows(r2, _):
            for u in range(2):
                r = r2 * 2 + u
                for j in range(_D // 16):
                    sl = pl.ds(j * 16, 16)
                    buf[r, sl] = buf[r, sl] + pos_v[r, sl]
            return ()

        lax.fori_loop(0, _S // 2, add_rows, ())
        start_store(s, b)

    for s in range(_LA):
        start_gather(s, s % _NB)
    for s in range(_NB - _LA):
        step(s, s % _NB, issue_next=True, wstore=False)
    ph = _NB - _LA

    def main(g, _):
        s0 = ph + g * _NB
        for b0 in range(_NB):
            step(s0 + b0, (ph + b0) % _NB, issue_next=True, wstore=True)
        return ()

    lax.fori_loop(0, (_SPW - _LA - ph) // _NB, main, ())
    for s in range(_SPW - _LA, _SPW):
        step(s, s % _NB, issue_next=False, wstore=False)
    for s in range(_SPW - _NB, _SPW):
        wait_store(s % _NB)


def _epilogue_body(packed_ref, out_ref):
    x = packed_ref[...]
    x3 = x.reshape(_SEQ_BLK, _H, 2 * _D)
    lo = x3[:, :, :_D]
    hi = x3[:, :, _D:]
    out_ref[...] = jnp.concatenate([lo, hi], axis=1)


@jax.jit
def _run(idx, table, pos):
    mesh = plsc.VectorSubcoreMesh(core_axis_name="c", subcore_axis_name="s")
    gather = pl.kernel(
        _gather_body,
        out_type=jax.ShapeDtypeStruct((_B * _H, 2 * _D), jnp.float32),
        mesh=mesh,
        scratch_types=[
            pltpu.VMEM((_SPW, _SP), jnp.int32),
            pltpu.VMEM((_S, _D), jnp.float32),
            [pltpu.VMEM((_S, _D), jnp.float32) for _ in range(_NB)],
            [pltpu.SemaphoreType.DMA for _ in range(_NB)],
            [pltpu.SemaphoreType.DMA for _ in range(_NB)],
        ],
        compiler_params=pltpu.CompilerParams(use_tc_tiling_on_sc=False),
    )
    packed = gather(table, idx, pos)

    return pl.pallas_call(
        _epilogue_body,
        out_shape=jax.ShapeDtypeStruct((_B, _S, _D), jnp.float32),
        grid=(_B // _SEQ_BLK,),
        in_specs=[pl.BlockSpec((_SEQ_BLK * _H, 2 * _D), lambda i: (i, 0))],
        out_specs=pl.BlockSpec((_SEQ_BLK, _S, _D), lambda i: (i, 0, 0)),
    )(packed)


def _make_pos(len_seq, embedding_dim):
    positions = jnp.arange(0.0, len_seq)[:, None]
    components_even_idx = jnp.arange(0.0, embedding_dim, 2)
    div = 10000.0 ** (components_even_idx / embedding_dim)
    pos = jnp.zeros((len_seq, embedding_dim), dtype=jnp.float32)
    pos = pos.at[:, 1::2].set(jnp.sin(positions / div))
    pos = pos.at[:, 0::2].set(jnp.cos(positions / div))
    return pos


def kernel(input, table):
    pos = _make_pos(_S, _D)
    idx = jnp.pad(input.astype(jnp.int32), ((0, 0), (0, _SP - _S)))
    out = _run(idx, table, pos)
    return lax.stop_gradient(out)


# speedup vs baseline: 1.1394x; 1.1394x over previous
import functools

import jax
import jax.numpy as jnp
from jax import lax
from jax.experimental import pallas as pl
from jax.experimental.pallas import tpu as pltpu
from jax.experimental.pallas import tpu_sc as plsc

_VOCAB = 100000
_D = 64
_S = 200
_SP = 256
_B = 1024
_H = _S // 2

_NC = 2
_NS = 16
_NW = _NC * _NS
_SPW = _B // _NW
_NB = 8
_LA = 4

_SEQ_BLK = 64

_SPLITS = ((0, 104), (104, 96))


def _gather_body(table_hbm, idx_hbm, pos_hbm, out_hbm,
                 idx_v, pos_v, bufs, gsems, ssems):
    wid = lax.axis_index("s") * _NC + lax.axis_index("c")
    sbase = wid * _SPW

    pltpu.sync_copy(pos_hbm, pos_v)
    pltpu.sync_copy(idx_hbm.at[pl.ds(wid * _SPW, _SPW), :], idx_v)

    def start_gather(s, b):
        buf = bufs[b]
        for o, n in _SPLITS:
            pltpu.make_async_copy(
                table_hbm.at[idx_v.at[s, pl.ds(o, n)]],
                buf.at[pl.ds(o, n)], gsems[b]).start()

    def wait_gather(s, b):
        buf = bufs[b]
        for o, n in _SPLITS:
            pltpu.make_async_copy(
                table_hbm.at[idx_v.at[s, pl.ds(o, n)]],
                buf.at[pl.ds(o, n)], gsems[b]).wait()

    def start_store(s, b):
        buf = bufs[b]
        r0 = (sbase + s) * _H
        for h in range(2):
            pltpu.make_async_copy(
                buf.at[pl.ds(h * _H, _H), :],
                out_hbm.at[pl.ds(r0, _H), pl.ds(h * _D, _D)], ssems[b]).start()

    def wait_store(b):
        buf = bufs[b]
        for h in range(2):
            pltpu.make_async_copy(
                buf.at[pl.ds(h * _H, _H), :],
                out_hbm.at[pl.ds(0, _H), pl.ds(h * _D, _D)], ssems[b]).wait()

    def step(s, b, issue_next, wstore):
        if issue_next:
            nb = (b + _LA) % _NB
            if wstore:
                wait_store(nb)
            start_gather(s + _LA, nb)
        wait_gather(s, b)
        buf = bufs[b]

        def add_rows(r2, _):
            for u in range(2):
                r = r2 * 2 + u
                for j in range(_D // 16):
                    sl = pl.ds(j * 16, 16)
                    buf[r, sl] = buf[r, sl] + pos_v[r, sl]
            return ()

        lax.fori_loop(0, _S // 2, add_rows, ())
        start_store(s, b)

    for s in range(_LA):
        start_gather(s, s % _NB)
    for s in range(_NB - _LA):
        step(s, s % _NB, issue_next=True, wstore=False)
    ph = _NB - _LA

    def main(g, _):
        s0 = ph + g * _NB
        for b0 in range(_NB):
            step(s0 + b0, (ph + b0) % _NB, issue_next=True, wstore=True)
        return ()

    lax.fori_loop(0, (_SPW - _LA - ph) // _NB, main, ())
    for s in range(_SPW - _LA, _SPW):
        step(s, s % _NB, issue_next=False, wstore=False)
    for s in range(_SPW - _NB, _SPW):
        wait_store(s % _NB)


def _epilogue_body(packed_ref, out_ref):
    x = packed_ref[...]
    x3 = x.reshape(_SEQ_BLK, _H, 2 * _D)
    lo = x3[:, :, :_D]
    hi = x3[:, :, _D:]
    y = jnp.concatenate([lo, hi], axis=1)
    out_ref[...] = y.reshape(_SEQ_BLK * _S, _D)


@jax.jit
def _run(idx, table, pos):
    mesh = plsc.VectorSubcoreMesh(core_axis_name="c", subcore_axis_name="s")
    gather = pl.kernel(
        _gather_body,
        out_type=jax.ShapeDtypeStruct((_B * _H, 2 * _D), jnp.float32),
        mesh=mesh,
        scratch_types=[
            pltpu.VMEM((_SPW, _SP), jnp.int32),
            pltpu.VMEM((_S, _D), jnp.float32),
            [pltpu.VMEM((_S, _D), jnp.float32) for _ in range(_NB)],
            [pltpu.SemaphoreType.DMA for _ in range(_NB)],
            [pltpu.SemaphoreType.DMA for _ in range(_NB)],
        ],
        compiler_params=pltpu.CompilerParams(use_tc_tiling_on_sc=False),
    )
    packed = gather(table, idx, pos)

    out2d = pl.pallas_call(
        _epilogue_body,
        out_shape=jax.ShapeDtypeStruct((_B * _S, _D), jnp.float32),
        grid=(_B // _SEQ_BLK,),
        in_specs=[pl.BlockSpec((_SEQ_BLK * _H, 2 * _D), lambda i: (i, 0))],
        out_specs=pl.BlockSpec((_SEQ_BLK * _S, _D), lambda i: (i, 0)),
    )(packed)
    return out2d.reshape(_B, _S, _D)


def _make_pos(len_seq, embedding_dim):
    positions = jnp.arange(0.0, len_seq)[:, None]
    components_even_idx = jnp.arange(0.0, embedding_dim, 2)
    div = 10000.0 ** (components_even_idx / embedding_dim)
    pos = jnp.zeros((len_seq, embedding_dim), dtype=jnp.float32)
    pos = pos.at[:, 1::2].set(jnp.sin(positions / div))
    pos = pos.at[:, 0::2].set(jnp.cos(positions / div))
    return pos


def kernel(input, table):
    pos = _make_pos(_S, _D)
    idx = jnp.pad(input.astype(jnp.int32), ((0, 0), (0, _SP - _S)))
    out = _run(idx, table, pos)
    return lax.stop_gradient(out)


# speedup vs baseline: 1.1462x; 1.0060x over previous
import functools

import jax
import jax.numpy as jnp
from jax import lax
from jax.experimental import pallas as pl
from jax.experimental.pallas import tpu as pltpu
from jax.experimental.pallas import tpu_sc as plsc

_VOCAB = 100000
_D = 64
_S = 200
_SP = 256
_B = 1024
_H = _S // 2

_NC = 2
_NS = 16
_NW = _NC * _NS
_SPW = _B // _NW
_NB = 8
_LA = 4

_SEQ_BLK = 128

_SPLITS = ((0, 104), (104, 96))


def _gather_body(table_hbm, idx_hbm, pos_hbm, out_hbm,
                 idx_v, pos_v, bufs, gsems, ssems):
    wid = lax.axis_index("s") * _NC + lax.axis_index("c")
    sbase = wid * _SPW

    pltpu.sync_copy(pos_hbm, pos_v)
    pltpu.sync_copy(idx_hbm.at[pl.ds(wid * _SPW, _SPW), :], idx_v)

    def start_gather(s, b):
        buf = bufs[b]
        for o, n in _SPLITS:
            pltpu.make_async_copy(
                table_hbm.at[idx_v.at[s, pl.ds(o, n)]],
                buf.at[pl.ds(o, n)], gsems[b]).start()

    def wait_gather(s, b):
        buf = bufs[b]
        for o, n in _SPLITS:
            pltpu.make_async_copy(
                table_hbm.at[idx_v.at[s, pl.ds(o, n)]],
                buf.at[pl.ds(o, n)], gsems[b]).wait()

    def start_store(s, b):
        buf = bufs[b]
        r0 = (sbase + s) * _H
        for h in range(2):
            pltpu.make_async_copy(
                buf.at[pl.ds(h * _H, _H), :],
                out_hbm.at[pl.ds(r0, _H), pl.ds(h * _D, _D)], ssems[b]).start()

    def wait_store(b):
        buf = bufs[b]
        for h in range(2):
            pltpu.make_async_copy(
                buf.at[pl.ds(h * _H, _H), :],
                out_hbm.at[pl.ds(0, _H), pl.ds(h * _D, _D)], ssems[b]).wait()

    def step(s, b, issue_next, wstore):
        if issue_next:
            nb = (b + _LA) % _NB
            if wstore:
                wait_store(nb)
            start_gather(s + _LA, nb)
        wait_gather(s, b)
        buf = bufs[b]

        def add_rows(r2, _):
            for u in range(2):
                r = r2 * 2 + u
                for j in range(_D // 16):
                    sl = pl.ds(j * 16, 16)
                    buf[r, sl] = buf[r, sl] + pos_v[r, sl]
            return ()

        lax.fori_loop(0, _S // 2, add_rows, ())
        start_store(s, b)

    for s in range(_LA):
        start_gather(s, s % _NB)
    for s in range(_NB - _LA):
        step(s, s % _NB, issue_next=True, wstore=False)
    ph = _NB - _LA

    def main(g, _):
        s0 = ph + g * _NB
        for b0 in range(_NB):
            step(s0 + b0, (ph + b0) % _NB, issue_next=True, wstore=True)
        return ()

    lax.fori_loop(0, (_SPW - _LA - ph) // _NB, main, ())
    for s in range(_SPW - _LA, _SPW):
        step(s, s % _NB, issue_next=False, wstore=False)
    for s in range(_SPW - _NB, _SPW):
        wait_store(s % _NB)


def _epilogue_body(packed_ref, out_ref):
    x = packed_ref[...]
    x3 = x.reshape(_SEQ_BLK, _H, 2 * _D)
    lo = x3[:, :, :_D]
    hi = x3[:, :, _D:]
    y = jnp.concatenate([lo, hi], axis=1)
    out_ref[...] = y.reshape(_SEQ_BLK * _S, _D)


@jax.jit
def _run(idx, table, pos):
    mesh = plsc.VectorSubcoreMesh(core_axis_name="c", subcore_axis_name="s")
    gather = pl.kernel(
        _gather_body,
        out_type=jax.ShapeDtypeStruct((_B * _H, 2 * _D), jnp.float32),
        mesh=mesh,
        scratch_types=[
            pltpu.VMEM((_SPW, _SP), jnp.int32),
            pltpu.VMEM((_S, _D), jnp.float32),
            [pltpu.VMEM((_S, _D), jnp.float32) for _ in range(_NB)],
            [pltpu.SemaphoreType.DMA for _ in range(_NB)],
            [pltpu.SemaphoreType.DMA for _ in range(_NB)],
        ],
        compiler_params=pltpu.CompilerParams(use_tc_tiling_on_sc=False),
    )
    packed = gather(table, idx, pos)

    out2d = pl.pallas_call(
        _epilogue_body,
        out_shape=jax.ShapeDtypeStruct((_B * _S, _D), jnp.float32),
        grid=(_B // _SEQ_BLK,),
        in_specs=[pl.BlockSpec((_SEQ_BLK * _H, 2 * _D), lambda i: (i, 0))],
        out_specs=pl.BlockSpec((_SEQ_BLK * _S, _D), lambda i: (i, 0)),
    )(packed)
    return out2d.reshape(_B, _S, _D)


def _make_pos(len_seq, embedding_dim):
    positions = jnp.arange(0.0, len_seq)[:, None]
    components_even_idx = jnp.arange(0.0, embedding_dim, 2)
    div = 10000.0 ** (components_even_idx / embedding_dim)
    pos = jnp.zeros((len_seq, embedding_dim), dtype=jnp.float32)
    pos = pos.at[:, 1::2].set(jnp.sin(positions / div))
    pos = pos.at[:, 0::2].set(jnp.cos(positions / div))
    return pos


def kernel(input, table):
    pos = _make_pos(_S, _D)
    idx = jnp.pad(input.astype(jnp.int32), ((0, 0), (0, _SP - _S)))
    out = _run(idx, table, pos)
    return lax.stop_gradient(out)
